# baseline (speedup 1.0000x reference)
import functools

import jax
import jax.numpy as jnp
from jax import lax
from jax.experimental import pallas as pl
from jax.experimental.pallas import tpu as pltpu
from jax.experimental.pallas import tpu_sc as plsc

B, T, D_IN, D_CODE, K, NCB = 16, 1024, 768, 256, 1024, 2
NTOK = B * T
BLK = 1024
SC_WORKERS = 32
SC_CHUNK = 128


def _first_argmin(dist):
    m = jnp.min(dist, axis=1, keepdims=True)
    iota = jax.lax.broadcasted_iota(jnp.int32, dist.shape, 1)
    idx = jnp.min(jnp.where(dist == m, iota, K), axis=1)
    return idx, iota


def _body_a(x_ref, we_ref, be_ref, cbt0_ref, c20_ref, enc_ref, idx_ref):
    x = x_ref[...]
    enc = jnp.dot(x, we_ref[...], preferred_element_type=jnp.float32) + be_ref[...]
    res = enc
    r2 = jnp.sum(res * res, axis=1, keepdims=True)
    dots = jnp.dot(res, cbt0_ref[...], preferred_element_type=jnp.float32)
    dist = r2 - 2.0 * dots + c20_ref[...]
    idx, _ = _first_argmin(dist)
    enc_ref[...] = enc
    idx_ref[...] = idx[None, None, :]


def _body_b(enc_ref, q1_ref, cb1_ref, cbt1_ref, c21_ref, wd_ref, bd_ref, o_ref):
    enc = enc_ref[...]
    q1 = q1_ref[...]
    res = enc - q1
    r2 = jnp.sum(res * res, axis=1, keepdims=True)
    dots = jnp.dot(res, cbt1_ref[...], preferred_element_type=jnp.float32)
    dist = r2 - 2.0 * dots + c21_ref[...]
    idx, iota = _first_argmin(dist)
    onehot = (iota == idx[:, None]).astype(jnp.float32)
    q2 = jnp.dot(onehot, cb1_ref[...], preferred_element_type=jnp.float32)
    s = q1 + q2
    codes = enc + (s - enc)
    o_ref[...] = jnp.dot(codes, wd_ref[...], preferred_element_type=jnp.float32) + bd_ref[...]


def _sc_gather(table, idx):
    n = idx.shape[0]
    bpw = n // SC_WORKERS
    mesh = plsc.VectorSubcoreMesh(core_axis_name="c", subcore_axis_name="s")

    @functools.partial(
        pl.kernel, mesh=mesh,
        out_type=jax.ShapeDtypeStruct((n, D_CODE), jnp.float32),
        scratch_types=[
            pltpu.VMEM((SC_CHUNK,), jnp.int32),
            pltpu.VMEM((SC_CHUNK, D_CODE), jnp.float32),
            pltpu.SemaphoreType.DMA,
        ],
    )
    def k(table_hbm, idx_hbm, out_hbm, idx_v, rows_v, sem):
        wid = lax.axis_index("s") * 2 + lax.axis_index("c")
        base = wid * bpw
        for c in range(bpw // SC_CHUNK):
            pos = base + c * SC_CHUNK
            pltpu.sync_copy(idx_hbm.at[pl.ds(pos, SC_CHUNK)], idx_v)
            pltpu.async_copy(table_hbm.at[idx_v], rows_v, sem).wait()
            pltpu.sync_copy(rows_v, out_hbm.at[pl.ds(pos, SC_CHUNK)])

    return k(table, idx)


@jax.jit
def _run(xf, W_enc, b_enc, W_dec, b_dec, codebooks, cbT, c2_all):
    n_blk = NTOK // BLK
    rep = lambda i: (0, 0)
    enc, idx1 = pl.pallas_call(
        _body_a,
        grid=(n_blk,),
        in_specs=[
            pl.BlockSpec((BLK, D_IN), lambda i: (i, 0)),
            pl.BlockSpec((D_IN, D_CODE), rep),
            pl.BlockSpec((1, D_CODE), rep),
            pl.BlockSpec((D_CODE, K), rep),
            pl.BlockSpec((1, K), rep),
        ],
        out_specs=[
            pl.BlockSpec((BLK, D_CODE), lambda i: (i, 0)),
            pl.BlockSpec((1, 1, BLK), lambda i: (i, 0, 0)),
        ],
        out_shape=[
            jax.ShapeDtypeStruct((NTOK, D_CODE), jnp.float32),
            jax.ShapeDtypeStruct((n_blk, 1, BLK), jnp.int32),
        ],
    )(xf, W_enc, b_enc.reshape(1, D_CODE), cbT[0], c2_all[0])

    q1 = _sc_gather(codebooks[0], idx1.reshape(NTOK))

    out = pl.pallas_call(
        _body_b,
        grid=(n_blk,),
        in_specs=[
            pl.BlockSpec((BLK, D_CODE), lambda i: (i, 0)),
            pl.BlockSpec((BLK, D_CODE), lambda i: (i, 0)),
            pl.BlockSpec((K, D_CODE), rep),
            pl.BlockSpec((D_CODE, K), rep),
            pl.BlockSpec((1, K), rep),
            pl.BlockSpec((D_CODE, D_IN), rep),
            pl.BlockSpec((1, D_IN), rep),
        ],
        out_specs=pl.BlockSpec((BLK, D_IN), lambda i: (i, 0)),
        out_shape=jax.ShapeDtypeStruct((NTOK, D_IN), jnp.float32),
    )(enc, q1, codebooks[1], cbT[1], c2_all[1], W_dec, b_dec.reshape(1, D_IN))
    return out


def kernel(x, W_enc, b_enc, W_dec, b_dec, codebooks):
    xf = x.reshape(NTOK, D_IN)
    cbT = jnp.transpose(codebooks, (0, 2, 1))
    c2_all = jnp.stack([jnp.sum(codebooks[i] ** 2, axis=-1)[None, :]
                        for i in range(NCB)])
    out = _run(xf, W_enc, b_enc, W_dec, b_dec, codebooks, cbT, c2_all)
    return out.reshape(B, T, D_IN)


# speedup vs baseline: 1.0062x; 1.0062x over previous
import functools

import jax
import jax.numpy as jnp
from jax import lax
from jax.experimental import pallas as pl
from jax.experimental.pallas import tpu as pltpu
from jax.experimental.pallas import tpu_sc as plsc

B, T, D_IN, D_CODE, K, NCB = 16, 1024, 768, 256, 1024, 2
NTOK = B * T
BLK = 1024
SC_WORKERS = 32
SC_CHUNK = 128


def _first_argmin(dist):
    m = jnp.min(dist, axis=1, keepdims=True)
    iota = jax.lax.broadcasted_iota(jnp.int32, dist.shape, 1)
    idx = jnp.min(jnp.where(dist == m, iota, K), axis=1)
    return idx, iota


def _body_a(x_ref, we_ref, be_ref, cbt0_ref, c20_ref, enc_ref, idx_ref):
    x = x_ref[...]
    enc = jnp.dot(x, we_ref[...], preferred_element_type=jnp.float32) + be_ref[...]
    res = enc
    r2 = jnp.sum(res * res, axis=1, keepdims=True)
    dots = jnp.dot(res, cbt0_ref[...], preferred_element_type=jnp.float32)
    dist = r2 - 2.0 * dots + c20_ref[...]
    idx, _ = _first_argmin(dist)
    enc_ref[...] = enc
    idx_ref[...] = idx[None, None, :]


def _body_b(enc_ref, q1_ref, cb1_ref, cbt1_ref, c21_ref, wd_ref, bd_ref, o_ref):
    enc = enc_ref[...]
    q1 = q1_ref[...]
    res = enc - q1
    r2 = jnp.sum(res * res, axis=1, keepdims=True)
    dots = jnp.dot(res, cbt1_ref[...], preferred_element_type=jnp.float32)
    dist = r2 - 2.0 * dots + c21_ref[...]
    idx, iota = _first_argmin(dist)
    onehot = (iota == idx[:, None]).astype(jnp.float32)
    q2 = jnp.dot(onehot, cb1_ref[...], preferred_element_type=jnp.float32)
    s = q1 + q2
    codes = enc + (s - enc)
    o_ref[...] = jnp.dot(codes, wd_ref[...], preferred_element_type=jnp.float32) + bd_ref[...]


def _sc_gather(table, idx):
    n = idx.shape[0]
    bpw = n // SC_WORKERS
    mesh = plsc.VectorSubcoreMesh(core_axis_name="c", subcore_axis_name="s")

    n_chunks = bpw // SC_CHUNK

    @functools.partial(
        pl.kernel, mesh=mesh,
        out_type=jax.ShapeDtypeStruct((n, D_CODE), jnp.float32),
        scratch_types=[
            pltpu.VMEM((bpw,), jnp.int32),
            pltpu.VMEM((SC_CHUNK, D_CODE), jnp.float32),
            pltpu.VMEM((SC_CHUNK, D_CODE), jnp.float32),
            pltpu.VMEM((SC_CHUNK, D_CODE), jnp.float32),
            pltpu.SemaphoreType.DMA,
            pltpu.SemaphoreType.DMA,
            pltpu.SemaphoreType.DMA,
            pltpu.SemaphoreType.DMA,
            pltpu.SemaphoreType.DMA,
            pltpu.SemaphoreType.DMA,
        ],
    )
    def k2(table_hbm, idx_hbm, out_hbm, idx_v, b0, b1, b2, g0, g1, g2,
           o0, o1, o2):
        wid = lax.axis_index("s") * 2 + lax.axis_index("c")
        base = wid * bpw
        pltpu.sync_copy(idx_hbm.at[pl.ds(base, bpw)], idx_v)
        bufs = (b0, b1, b2)
        gsem = (g0, g1, g2)
        osem = (o0, o1, o2)
        gats = [None] * n_chunks
        wrs = [None] * n_chunks
        for c in range(n_chunks):
            s = c % 3
            if c >= 3:
                wrs[c - 3].wait()
            gats[c] = pltpu.async_copy(
                table_hbm.at[idx_v.at[pl.ds(c * SC_CHUNK, SC_CHUNK)]],
                bufs[s], gsem[s])
            if c >= 1:
                p = c - 1
                gats[p].wait()
                wrs[p] = pltpu.async_copy(
                    bufs[p % 3],
                    out_hbm.at[pl.ds(base + p * SC_CHUNK, SC_CHUNK)],
                    osem[p % 3])
        last = n_chunks - 1
        gats[last].wait()
        wrs[last] = pltpu.async_copy(
            bufs[last % 3],
            out_hbm.at[pl.ds(base + last * SC_CHUNK, SC_CHUNK)],
            osem[last % 3])
        for p in range(max(0, n_chunks - 3), n_chunks):
            wrs[p].wait()

    return k2(table, idx)


@jax.jit
def _run(xf, W_enc, b_enc, W_dec, b_dec, codebooks, cbT, c2_all):
    n_blk = NTOK // BLK
    rep = lambda i: (0, 0)
    enc, idx1 = pl.pallas_call(
        _body_a,
        grid=(n_blk,),
        in_specs=[
            pl.BlockSpec((BLK, D_IN), lambda i: (i, 0)),
            pl.BlockSpec((D_IN, D_CODE), rep),
            pl.BlockSpec((1, D_CODE), rep),
            pl.BlockSpec((D_CODE, K), rep),
            pl.BlockSpec((1, K), rep),
        ],
        out_specs=[
            pl.BlockSpec((BLK, D_CODE), lambda i: (i, 0)),
            pl.BlockSpec((1, 1, BLK), lambda i: (i, 0, 0)),
        ],
        out_shape=[
            jax.ShapeDtypeStruct((NTOK, D_CODE), jnp.float32),
            jax.ShapeDtypeStruct((n_blk, 1, BLK), jnp.int32),
        ],
    )(xf, W_enc, b_enc.reshape(1, D_CODE), cbT[0], c2_all[0])

    q1 = _sc_gather(codebooks[0], idx1.reshape(NTOK))

    out = pl.pallas_call(
        _body_b,
        grid=(n_blk,),
        in_specs=[
            pl.BlockSpec((BLK, D_CODE), lambda i: (i, 0)),
            pl.BlockSpec((BLK, D_CODE), lambda i: (i, 0)),
            pl.BlockSpec((K, D_CODE), rep),
            pl.BlockSpec((D_CODE, K), rep),
            pl.BlockSpec((1, K), rep),
            pl.BlockSpec((D_CODE, D_IN), rep),
            pl.BlockSpec((1, D_IN), rep),
        ],
        out_specs=pl.BlockSpec((BLK, D_IN), lambda i: (i, 0)),
        out_shape=jax.ShapeDtypeStruct((NTOK, D_IN), jnp.float32),
    )(enc, q1, codebooks[1], cbT[1], c2_all[1], W_dec, b_dec.reshape(1, D_IN))
    return out


def kernel(x, W_enc, b_enc, W_dec, b_dec, codebooks):
    xf = x.reshape(NTOK, D_IN)
    cbT = jnp.transpose(codebooks, (0, 2, 1))
    c2_all = jnp.stack([jnp.sum(codebooks[i] ** 2, axis=-1)[None, :]
                        for i in range(NCB)])
    out = _run(xf, W_enc, b_enc, W_dec, b_dec, codebooks, cbT, c2_all)
    return out.reshape(B, T, D_IN)


# speedup vs baseline: 1.2060x; 1.1986x over previous
import functools

import jax
import jax.numpy as jnp
from jax.experimental import pallas as pl

B, T, D_IN, D_CODE, K, NCB = 16, 1024, 768, 256, 1024, 2
BLK = 1024


def _body(x_ref, we_ref, be_ref, cb_ref, cbt_ref, wd_ref, bd_ref, cb0s_ref,
          c2_ref, o_ref):
    x = x_ref[...]
    enc = jnp.dot(x, we_ref[...], preferred_element_type=jnp.float32) + be_ref[...]
    res = enc
    qsum = jnp.zeros_like(enc)
    for i in range(NCB):
        cb = cb_ref[i]
        cbt = cbt_ref[i]
        r2 = jnp.sum(res * res, axis=1, keepdims=True)
        c2 = c2_ref[i]
        dots = jnp.dot(res, cbt, preferred_element_type=jnp.float32)
        dist = r2 - 2.0 * dots + c2
        iota = jax.lax.broadcasted_iota(jnp.int32, dist.shape, 1)
        idx = jnp.argmin(dist, axis=1).astype(jnp.int32)
        onehot = (iota == idx[:, None]).astype(jnp.float32)
        if i == 0:
            q = (jnp.dot(onehot, cb0s_ref[0], preferred_element_type=jnp.float32)
                 + jnp.dot(onehot, cb0s_ref[1], preferred_element_type=jnp.float32)
                 + jnp.dot(onehot, cb0s_ref[2], preferred_element_type=jnp.float32))
        else:
            q = jnp.dot(onehot, cb, preferred_element_type=jnp.float32)
        qsum = qsum + q
        res = res - q
    codes = enc + (qsum - enc)
    o_ref[...] = jnp.dot(codes, wd_ref[...], preferred_element_type=jnp.float32) + bd_ref[...]


@jax.jit
def _run(xf, W_enc, b_enc, W_dec, b_dec, codebooks, cbT, cb0_split, c2_all):
    n_blk = (B * T) // BLK
    return pl.pallas_call(
        _body,
        grid=(n_blk,),
        in_specs=[
            pl.BlockSpec((BLK, D_IN), lambda i: (i, 0)),
            pl.BlockSpec((D_IN, D_CODE), lambda i: (0, 0)),
            pl.BlockSpec((1, D_CODE), lambda i: (0, 0)),
            pl.BlockSpec((NCB, K, D_CODE), lambda i: (0, 0, 0)),
            pl.BlockSpec((NCB, D_CODE, K), lambda i: (0, 0, 0)),
            pl.BlockSpec((D_CODE, D_IN), lambda i: (0, 0)),
            pl.BlockSpec((1, D_IN), lambda i: (0, 0)),
            pl.BlockSpec((3, K, D_CODE), lambda i: (0, 0, 0)),
            pl.BlockSpec((NCB, 1, K), lambda i: (0, 0, 0)),
        ],
        out_specs=pl.BlockSpec((BLK, D_IN), lambda i: (i, 0)),
        out_shape=jax.ShapeDtypeStruct((B * T, D_IN), jnp.float32),
    )(xf, W_enc, b_enc.reshape(1, D_CODE), codebooks, cbT, W_dec,
      b_dec.reshape(1, D_IN), cb0_split, c2_all)


def _split3(a):
    def trunc(v):
        bits = jax.lax.bitcast_convert_type(v, jnp.uint32)
        return jax.lax.bitcast_convert_type(bits & jnp.uint32(0xFFFF0000),
                                            jnp.float32)
    hi = trunc(a)
    r1 = a - hi
    mid = trunc(r1)
    r2 = r1 - mid
    lo = trunc(r2)
    return jnp.stack([hi, mid, lo])


def kernel(x, W_enc, b_enc, W_dec, b_dec, codebooks):
    xf = x.reshape(B * T, D_IN)
    cbT = jnp.transpose(codebooks, (0, 2, 1))
    cb0_split = _split3(codebooks[0])
    c2_all = jnp.stack([jnp.sum(codebooks[i] ** 2, axis=-1)[None, :]
                        for i in range(NCB)])
    out = _run(xf, W_enc, b_enc, W_dec, b_dec, codebooks, cbT, cb0_split,
               c2_all)
    return out.reshape(B, T, D_IN)


# speedup vs baseline: 1.2886x; 1.0685x over previous
import functools

import jax
import jax.numpy as jnp
from jax.experimental import pallas as pl

B, T, D_IN, D_CODE, K, NCB = 16, 1024, 768, 256, 1024, 2
BLK = 2048


def _body(x_ref, we_ref, be_ref, cb_ref, cbt_ref, wd_ref, bd_ref, cb0s_ref,
          c2_ref, o_ref):
    x = x_ref[...]
    enc = jnp.dot(x, we_ref[...], preferred_element_type=jnp.float32) + be_ref[...]
    res = enc
    qsum = jnp.zeros_like(enc)
    for i in range(NCB):
        cb = cb_ref[i]
        cbt = cbt_ref[i]
        r2 = jnp.sum(res * res, axis=1, keepdims=True)
        c2 = c2_ref[i]
        dots = jnp.dot(res, cbt, preferred_element_type=jnp.float32)
        dist = r2 - 2.0 * dots + c2
        iota = jax.lax.broadcasted_iota(jnp.int32, dist.shape, 1)
        idx = jnp.argmin(dist, axis=1).astype(jnp.int32)
        onehot = (iota == idx[:, None]).astype(jnp.float32)
        if i == 0:
            q = (jnp.dot(onehot, cb0s_ref[0], preferred_element_type=jnp.float32)
                 + jnp.dot(onehot, cb0s_ref[1], preferred_element_type=jnp.float32)
                 + jnp.dot(onehot, cb0s_ref[2], preferred_element_type=jnp.float32))
        else:
            q = jnp.dot(onehot, cb, preferred_element_type=jnp.float32)
        qsum = qsum + q
        res = res - q
    codes = enc + (qsum - enc)
    o_ref[...] = jnp.dot(codes, wd_ref[...], preferred_element_type=jnp.float32) + bd_ref[...]


@jax.jit
def _run(xf, W_enc, b_enc, W_dec, b_dec, codebooks, cbT, cb0_split, c2_all):
    n_blk = (B * T) // BLK
    return pl.pallas_call(
        _body,
        grid=(n_blk,),
        in_specs=[
            pl.BlockSpec((BLK, D_IN), lambda i: (i, 0)),
            pl.BlockSpec((D_IN, D_CODE), lambda i: (0, 0)),
            pl.BlockSpec((1, D_CODE), lambda i: (0, 0)),
            pl.BlockSpec((NCB, K, D_CODE), lambda i: (0, 0, 0)),
            pl.BlockSpec((NCB, D_CODE, K), lambda i: (0, 0, 0)),
            pl.BlockSpec((D_CODE, D_IN), lambda i: (0, 0)),
            pl.BlockSpec((1, D_IN), lambda i: (0, 0)),
            pl.BlockSpec((3, K, D_CODE), lambda i: (0, 0, 0)),
            pl.BlockSpec((NCB, 1, K), lambda i: (0, 0, 0)),
        ],
        out_specs=pl.BlockSpec((BLK, D_IN), lambda i: (i, 0)),
        out_shape=jax.ShapeDtypeStruct((B * T, D_IN), jnp.float32),
    )(xf, W_enc, b_enc.reshape(1, D_CODE), codebooks, cbT, W_dec,
      b_dec.reshape(1, D_IN), cb0_split, c2_all)


def _split3(a):
    def trunc(v):
        bits = jax.lax.bitcast_convert_type(v, jnp.uint32)
        return jax.lax.bitcast_convert_type(bits & jnp.uint32(0xFFFF0000),
                                            jnp.float32)
    hi = trunc(a)
    r1 = a - hi
    mid = trunc(r1)
    r2 = r1 - mid
    lo = trunc(r2)
    return jnp.stack([hi, mid, lo])


def kernel(x, W_enc, b_enc, W_dec, b_dec, codebooks):
    xf = x.reshape(B * T, D_IN)
    cbT = jnp.transpose(codebooks, (0, 2, 1))
    cb0_split = _split3(codebooks[0])
    c2_all = jnp.stack([jnp.sum(codebooks[i] ** 2, axis=-1)[None, :]
                        for i in range(NCB)])
    out = _run(xf, W_enc, b_enc, W_dec, b_dec, codebooks, cbT, cb0_split,
               c2_all)
    return out.reshape(B, T, D_IN)
